# baseline (speedup 1.0000x reference)
import jax
import jax.numpy as jnp
from jax.experimental import pallas as pl
from jax.experimental.pallas import tpu as pltpu
from jax.experimental.pallas import tpu_sc as plsc

D = 1024
DH = 512
FF = 512
E = 8
TILE = 256
SC_W = 128


def _dot_t(a, b):
    return jax.lax.dot_general(
        a.astype(jnp.bfloat16), b.astype(jnp.bfloat16),
        (((1,), (1,)), ((), ())), preferred_element_type=jnp.float32)


def _swiglu(xb, w1, w3, w2):
    h = jax.nn.silu(_dot_t(xb, w1)) * _dot_t(xb, w3)
    return _dot_t(h, w2)


def _grouped_body(te_ref, nu_ref, xa_ref, xb_ref, r1_ref, r3_ref, r2_ref,
                  ya_ref, yb_ref):
    i = pl.program_id(0)

    @pl.when(i < nu_ref[0])
    def _():
        xb = jnp.concatenate([xa_ref[...], xb_ref[...]], axis=-1)
        y = _swiglu(xb, r1_ref[0], r3_ref[0], r2_ref[0]).astype(jnp.bfloat16)
        ya_ref[...] = y[:, :DH]
        yb_ref[...] = y[:, DH:]


def _shared_body(x_ref, s1_ref, s3_ref, s2_ref, o_ref):
    o_ref[...] = _swiglu(x_ref[...], s1_ref[0], s3_ref[0], s2_ref[0])


def _combine_body(ys_ref, y1a_ref, y1b_ref, y2a_ref, y2b_ref,
                  w1_ref, w2_ref, o_ref):
    y1 = jnp.concatenate([y1a_ref[...], y1b_ref[...]], axis=-1)
    y2 = jnp.concatenate([y2a_ref[...], y2b_ref[...]], axis=-1)
    o_ref[...] = (ys_ref[...]
                  + w1_ref[0] * y1.astype(jnp.float32)
                  + w2_ref[0] * y2.astype(jnp.float32)) * (1.0 / 3.0)


def _pack_i32(a):
    n, c = a.shape
    return jax.lax.bitcast_convert_type(a.reshape(n, c // 2, 2), jnp.int32)


def _unpack_bf16(a):
    n, c = a.shape
    return jax.lax.bitcast_convert_type(a, jnp.bfloat16).reshape(n, 2 * c)


def _sc_scatter_half(x_half, idx, npad):
    T, dq = x_half.shape
    t2 = idx.shape[1]
    mesh = plsc.VectorSubcoreMesh(core_axis_name="c", subcore_axis_name="s")

    @pl.kernel(out_type=jax.ShapeDtypeStruct((npad, dq), jnp.int32),
               mesh=mesh)
    def k(x_hbm, i_hbm, o_hbm):
        def body(x_vmem, i_vmem):
            pltpu.sync_copy(x_vmem, o_hbm.at[i_vmem.at[0]])

        pltpu.emit_pipeline(
            body,
            grid=(t2 // SC_W,),
            in_specs=[
                pl.BlockSpec((SC_W, dq), lambda i: (i % (T // SC_W), 0)),
                pl.BlockSpec((1, SC_W), lambda i: (0, i)),
            ],
            out_specs=[],
            core_axis_name="s",
            dimension_semantics=(pltpu.PARALLEL,),
        )(x_hbm, i_hbm)

    return k(x_half, idx)


def _sc_gather_half(ys_half, idx):
    dq = ys_half.shape[1]
    t2 = idx.shape[1]
    mesh = plsc.VectorSubcoreMesh(core_axis_name="c", subcore_axis_name="s")

    @pl.kernel(out_type=jax.ShapeDtypeStruct((t2, dq), jnp.int32),
               mesh=mesh)
    def k(ys_hbm, i_hbm, o_hbm):
        def body(i_vmem, o_vmem):
            pltpu.sync_copy(ys_hbm.at[i_vmem.at[0]], o_vmem)

        pltpu.emit_pipeline(
            body,
            grid=(t2 // SC_W,),
            in_specs=[pl.BlockSpec((1, SC_W), lambda i: (0, i))],
            out_specs=[pl.BlockSpec((SC_W, dq), lambda i: (i, 0))],
            core_axis_name="s",
            dimension_semantics=(pltpu.PARALLEL,),
        )(i_hbm, o_hbm)

    return k(ys_half, idx)


def kernel(x, Wg, Ws1, Ws2, Ws3, Wr1, Wr2, Wr3):
    bs, seq_len, d = x.shape
    x_flat = x.reshape(-1, d)
    T = x_flat.shape[0]
    npad = 2 * T + E * TILE
    ntiles = npad // TILE

    gates = jax.nn.sigmoid(x_flat @ Wg.T)
    top_k_vals, top_k_indices = jax.lax.top_k(gates, 2)
    top_k_vals = top_k_vals / jnp.sum(top_k_vals, axis=-1, keepdims=True)

    m = (jax.nn.one_hot(top_k_indices[:, 0], E, dtype=jnp.int32)
         + jax.nn.one_hot(top_k_indices[:, 1], E, dtype=jnp.int32))
    pos = jnp.cumsum(m, axis=0) - m
    counts = jnp.sum(m, axis=0)
    padded = ((counts + TILE - 1) // TILE) * TILE
    seg_end = jnp.cumsum(padded)
    off = seg_end - padded
    pos_k = jnp.take_along_axis(pos, top_k_indices, axis=1)
    slots = off[top_k_indices] + pos_k
    slots_flat = jnp.concatenate([slots[:, 0], slots[:, 1]]).astype(jnp.int32)
    idx2 = slots_flat.reshape(1, 2 * T)
    tile_starts = jnp.arange(ntiles, dtype=jnp.int32) * TILE
    tile_expert = jnp.minimum(
        jnp.searchsorted(seg_end, tile_starts, side="right"), E - 1
    ).astype(jnp.int32)
    n_used = ((seg_end[-1] + TILE - 1) // TILE).astype(jnp.int32).reshape(1)

    x_bf = x_flat.astype(jnp.bfloat16)
    xp_a = _pack_i32(x_bf[:, :DH])
    xp_b = _pack_i32(x_bf[:, DH:])
    xs_a = _unpack_bf16(_sc_scatter_half(xp_a, idx2, npad))
    xs_b = _unpack_bf16(_sc_scatter_half(xp_b, idx2, npad))

    TT = 1024
    NT = T // TT
    y_shared = pl.pallas_call(
        _shared_body,
        grid=(NT,),
        in_specs=[
            pl.BlockSpec((TT, D), lambda t: (t, 0)),
            pl.BlockSpec((1, FF, D), lambda t: (0, 0, 0)),
            pl.BlockSpec((1, FF, D), lambda t: (0, 0, 0)),
            pl.BlockSpec((1, D, FF), lambda t: (0, 0, 0)),
        ],
        out_specs=pl.BlockSpec((TT, D), lambda t: (t, 0)),
        out_shape=jax.ShapeDtypeStruct((T, D), jnp.float32),
    )(x_flat, Ws1, Ws3, Ws2)

    grid_spec = pltpu.PrefetchScalarGridSpec(
        num_scalar_prefetch=2,
        grid=(ntiles,),
        in_specs=[
            pl.BlockSpec((TILE, DH), lambda i, te, nu: (i, 0)),
            pl.BlockSpec((TILE, DH), lambda i, te, nu: (i, 0)),
            pl.BlockSpec((1, FF, D), lambda i, te, nu: (te[i], 0, 0)),
            pl.BlockSpec((1, FF, D), lambda i, te, nu: (te[i], 0, 0)),
            pl.BlockSpec((1, D, FF), lambda i, te, nu: (te[i], 0, 0)),
        ],
        out_specs=[
            pl.BlockSpec((TILE, DH), lambda i, te, nu: (i, 0)),
            pl.BlockSpec((TILE, DH), lambda i, te, nu: (i, 0)),
        ],
    )
    ys_a, ys_b = pl.pallas_call(
        _grouped_body,
        grid_spec=grid_spec,
        out_shape=[
            jax.ShapeDtypeStruct((npad, DH), jnp.bfloat16),
            jax.ShapeDtypeStruct((npad, DH), jnp.bfloat16),
        ],
    )(tile_expert, n_used, xs_a, xs_b, Wr1, Wr3, Wr2)

    y12_a = _unpack_bf16(_sc_gather_half(_pack_i32(ys_a), idx2))
    y12_b = _unpack_bf16(_sc_gather_half(_pack_i32(ys_b), idx2))

    w1 = top_k_vals[:, 0].reshape(NT, TT, 1)
    w2 = top_k_vals[:, 1].reshape(NT, TT, 1)
    out = pl.pallas_call(
        _combine_body,
        grid=(NT,),
        in_specs=[
            pl.BlockSpec((TT, D), lambda t: (t, 0)),
            pl.BlockSpec((TT, DH), lambda t: (t, 0)),
            pl.BlockSpec((TT, DH), lambda t: (t, 0)),
            pl.BlockSpec((TT, DH), lambda t: (t + NT, 0)),
            pl.BlockSpec((TT, DH), lambda t: (t + NT, 0)),
            pl.BlockSpec((1, TT, 1), lambda t: (t, 0, 0)),
            pl.BlockSpec((1, TT, 1), lambda t: (t, 0, 0)),
        ],
        out_specs=pl.BlockSpec((TT, D), lambda t: (t, 0)),
        out_shape=jax.ShapeDtypeStruct((T, D), jnp.float32),
    )(y_shared, y12_a, y12_b, y12_a, y12_b, w1, w2)

    return out.reshape(bs, seq_len, d)


# speedup vs baseline: 6.0204x; 6.0204x over previous
import jax
import jax.numpy as jnp
from jax.experimental import pallas as pl
from jax.experimental.pallas import tpu as pltpu

D_MODEL = 1024
FF_DIM = 512
N_ROUTED = 8
N_EXPERTS = 9
INV_DENOM = 1.0 / 3.0


def _dot_t(a, b):
    return jax.lax.dot_general(
        a.astype(jnp.bfloat16), b.astype(jnp.bfloat16),
        (((1,), (1,)), ((), ())),
        preferred_element_type=jnp.float32,
    )


def _swiglu(xb, w1, w3, w2):
    h = jax.nn.silu(_dot_t(xb, w1)) * _dot_t(xb, w3)
    return _dot_t(h, w2)


def _moe_body(x_ref, s1_ref, s3_ref, s2_ref, r1_ref, r3_ref, r2_ref,
              sc_ref, o_ref):
    e = pl.program_id(1)
    xb = x_ref[...]

    @pl.when(e == 0)
    def _shared():
        y = _swiglu(xb, s1_ref[0], s3_ref[0], s2_ref[0])
        o_ref[...] = y * INV_DENOM

    @pl.when(e > 0)
    def _routed():
        y = _swiglu(xb, r1_ref[0], r3_ref[0], r2_ref[0])
        o_ref[...] = o_ref[...] + y * sc_ref[0, 0]


def kernel(x, Wg, Ws1, Ws2, Ws3, Wr1, Wr2, Wr3):
    bs, seq_len, d = x.shape
    x_flat = x.reshape(-1, d)
    T = x_flat.shape[0]

    gates = jax.nn.sigmoid(x_flat @ Wg.T)
    top_k_vals, top_k_indices = jax.lax.top_k(gates, 2)
    top_k_vals = top_k_vals / jnp.sum(top_k_vals, axis=-1, keepdims=True)
    weights = jnp.zeros((T, N_ROUTED), dtype=x_flat.dtype)
    for k in range(2):
        mask = jax.nn.one_hot(top_k_indices[:, k], N_ROUTED, dtype=x_flat.dtype)
        weights = weights + top_k_vals[:, k:k + 1] * mask

    x_bf = x_flat.astype(jnp.bfloat16)
    TT = 2048
    NT = T // TT
    scales = (weights.T * INV_DENOM).reshape(N_ROUTED, NT, TT, 1)

    re_idx = lambda t, e: (jnp.maximum(e - 1, 0), 0, 0)
    out = pl.pallas_call(
        _moe_body,
        grid=(NT, N_EXPERTS),
        in_specs=[
            pl.BlockSpec((TT, d), lambda t, e: (t, 0)),
            pl.BlockSpec((1, FF_DIM, d), lambda t, e: (0, 0, 0)),
            pl.BlockSpec((1, FF_DIM, d), lambda t, e: (0, 0, 0)),
            pl.BlockSpec((1, d, FF_DIM), lambda t, e: (0, 0, 0)),
            pl.BlockSpec((1, FF_DIM, d), re_idx),
            pl.BlockSpec((1, FF_DIM, d), re_idx),
            pl.BlockSpec((1, d, FF_DIM), re_idx),
            pl.BlockSpec((1, 1, TT, 1),
                         lambda t, e: (jnp.maximum(e - 1, 0), t, 0, 0)),
        ],
        out_specs=pl.BlockSpec((TT, d), lambda t, e: (t, 0)),
        out_shape=jax.ShapeDtypeStruct((T, d), jnp.float32),
    )(x_bf, Ws1, Ws3, Ws2, Wr1, Wr3, Wr2, scales)

    return out.reshape(bs, seq_len, d)


# speedup vs baseline: 6.1116x; 1.0151x over previous
import jax
import jax.numpy as jnp
from jax.experimental import pallas as pl
from jax.experimental.pallas import tpu as pltpu

D_MODEL = 1024
FF_DIM = 512
N_ROUTED = 8
N_EXPERTS = 9
INV_DENOM = 1.0 / 3.0


def _dot_t(a, b):
    return jax.lax.dot_general(
        a.astype(jnp.bfloat16), b.astype(jnp.bfloat16),
        (((1,), (1,)), ((), ())),
        preferred_element_type=jnp.float32,
    )


def _swiglu(xb, w1, w3, w2):
    h = jax.nn.silu(_dot_t(xb, w1)) * _dot_t(xb, w3)
    return _dot_t(h, w2)


def _moe_body(x_ref, s1_ref, s3_ref, s2_ref, r1_ref, r3_ref, r2_ref,
              sc_ref, o_ref):
    e = pl.program_id(1)
    xb = x_ref[...]

    @pl.when(e == 0)
    def _shared():
        y = _swiglu(xb, s1_ref[0], s3_ref[0], s2_ref[0])
        o_ref[...] = y * INV_DENOM

    @pl.when(e > 0)
    def _routed():
        y = _swiglu(xb, r1_ref[0], r3_ref[0], r2_ref[0])
        o_ref[...] = o_ref[...] + y * sc_ref[0, 0]


def kernel(x, Wg, Ws1, Ws2, Ws3, Wr1, Wr2, Wr3):
    bs, seq_len, d = x.shape
    x_flat = x.reshape(-1, d)
    T = x_flat.shape[0]

    gates = jax.nn.sigmoid(x_flat @ Wg.T)
    top_k_vals, top_k_indices = jax.lax.top_k(gates, 2)
    top_k_vals = top_k_vals / jnp.sum(top_k_vals, axis=-1, keepdims=True)
    weights = jnp.zeros((T, N_ROUTED), dtype=x_flat.dtype)
    for k in range(2):
        mask = jax.nn.one_hot(top_k_indices[:, k], N_ROUTED, dtype=x_flat.dtype)
        weights = weights + top_k_vals[:, k:k + 1] * mask

    x_bf = x_flat.astype(jnp.bfloat16)
    TT = 1024
    NT = T // TT
    scales = (weights.T * INV_DENOM).reshape(N_ROUTED, NT, TT, 1)

    re_idx = lambda t, e: (jnp.maximum(e - 1, 0), 0, 0)
    out = pl.pallas_call(
        _moe_body,
        grid=(NT, N_EXPERTS),
        in_specs=[
            pl.BlockSpec((TT, d), lambda t, e: (t, 0)),
            pl.BlockSpec((1, FF_DIM, d), lambda t, e: (0, 0, 0)),
            pl.BlockSpec((1, FF_DIM, d), lambda t, e: (0, 0, 0)),
            pl.BlockSpec((1, d, FF_DIM), lambda t, e: (0, 0, 0)),
            pl.BlockSpec((1, FF_DIM, d), re_idx),
            pl.BlockSpec((1, FF_DIM, d), re_idx),
            pl.BlockSpec((1, d, FF_DIM), re_idx),
            pl.BlockSpec((1, 1, TT, 1),
                         lambda t, e: (jnp.maximum(e - 1, 0), t, 0, 0)),
        ],
        out_specs=pl.BlockSpec((TT, d), lambda t, e: (t, 0)),
        out_shape=jax.ShapeDtypeStruct((T, d), jnp.float32),
    )(x_bf, Ws1, Ws3, Ws2, Wr1, Wr3, Wr2, scales)

    return out.reshape(bs, seq_len, d)


# speedup vs baseline: 6.5662x; 1.0744x over previous
import jax
import jax.numpy as jnp
from jax.experimental import pallas as pl
from jax.experimental.pallas import tpu as pltpu

D_MODEL = 1024
FF_DIM = 512
N_ROUTED = 8
N_EXPERTS = 9
INV_DENOM = 1.0 / 3.0


def _dot_t(a, b):
    return jax.lax.dot_general(
        a.astype(jnp.bfloat16), b.astype(jnp.bfloat16),
        (((1,), (1,)), ((), ())),
        preferred_element_type=jnp.float32,
    )


def _swiglu(xb, w1, w3, w2):
    h = jax.nn.silu(_dot_t(xb, w1)) * _dot_t(xb, w3)
    return _dot_t(h, w2)


def _moe_body(x_ref, s1_ref, s3_ref, s2_ref, r1_ref, r3_ref, r2_ref,
              sc_ref, o_ref, xbf_ref):
    e = pl.program_id(1)

    @pl.when(e == 0)
    def _shared():
        xbf_ref[...] = x_ref[...].astype(jnp.bfloat16)
        y = _swiglu(xbf_ref[...], s1_ref[0], s3_ref[0], s2_ref[0])
        o_ref[...] = y * INV_DENOM

    @pl.when(e > 0)
    def _routed():
        y = _swiglu(xbf_ref[...], r1_ref[0], r3_ref[0], r2_ref[0])
        o_ref[...] = o_ref[...] + y * sc_ref[0, 0]


def kernel(x, Wg, Ws1, Ws2, Ws3, Wr1, Wr2, Wr3):
    bs, seq_len, d = x.shape
    x_flat = x.reshape(-1, d)
    T = x_flat.shape[0]

    gates = jax.nn.sigmoid(x_flat @ Wg.T)
    top_k_vals, top_k_indices = jax.lax.top_k(gates, 2)
    top_k_vals = top_k_vals / jnp.sum(top_k_vals, axis=-1, keepdims=True)
    weights = jnp.zeros((T, N_ROUTED), dtype=x_flat.dtype)
    for k in range(2):
        mask = jax.nn.one_hot(top_k_indices[:, k], N_ROUTED, dtype=x_flat.dtype)
        weights = weights + top_k_vals[:, k:k + 1] * mask

    TT = 1024
    NT = T // TT
    scales = (weights.T * INV_DENOM).reshape(N_ROUTED, NT, TT, 1)

    re_idx = lambda t, e: (jnp.maximum(e - 1, 0), 0, 0)
    out = pl.pallas_call(
        _moe_body,
        grid=(NT, N_EXPERTS),
        in_specs=[
            pl.BlockSpec((TT, d), lambda t, e: (t, 0)),
            pl.BlockSpec((1, FF_DIM, d), lambda t, e: (0, 0, 0)),
            pl.BlockSpec((1, FF_DIM, d), lambda t, e: (0, 0, 0)),
            pl.BlockSpec((1, d, FF_DIM), lambda t, e: (0, 0, 0)),
            pl.BlockSpec((1, FF_DIM, d), re_idx),
            pl.BlockSpec((1, FF_DIM, d), re_idx),
            pl.BlockSpec((1, d, FF_DIM), re_idx),
            pl.BlockSpec((1, 1, TT, 1),
                         lambda t, e: (jnp.maximum(e - 1, 0), t, 0, 0)),
        ],
        out_specs=pl.BlockSpec((TT, d), lambda t, e: (t, 0)),
        out_shape=jax.ShapeDtypeStruct((T, d), jnp.float32),
        scratch_shapes=[pltpu.VMEM((TT, d), jnp.bfloat16)],
    )(x_flat, Ws1, Ws3, Ws2, Wr1, Wr3, Wr2, scales)

    return out.reshape(bs, seq_len, d)
